# Initial kernel scaffold; baseline (speedup 1.0000x reference)
import functools

import jax
import jax.numpy as jnp
from jax import lax
from jax.experimental import pallas as pl
from jax.experimental.pallas import tpu as pltpu
from jax.experimental.pallas import tpu_sc as plsc

VOCAB = 1000000
EMBED = 64
BATCH = 16384
NEG = 5
NIDX = NEG + 1

NC, NS, LANES = 2, 16, 16
NW = NC * NS
BPW = BATCH // NW
CB = 128
NCHUNK = BPW // CB
NGROUP = CB // LANES


def _sc_scores(pos_w, vidx, w_table, v_table):
    mesh = plsc.VectorSubcoreMesh(core_axis_name="c", subcore_axis_name="s")

    @functools.partial(
        pl.kernel,
        out_type=jax.ShapeDtypeStruct((NW, NCHUNK, NIDX, CB), jnp.float32),
        mesh=mesh,
        scratch_types=[
            pltpu.VMEM((CB,), jnp.int32),
            pltpu.VMEM((NIDX, CB), jnp.int32),
            pltpu.VMEM((CB, EMBED), jnp.float32),
            pltpu.VMEM((NIDX, CB, EMBED), jnp.float32),
            pltpu.VMEM((NIDX, CB), jnp.float32),
            pltpu.SemaphoreType.DMA,
        ],
    )
    def k(pos_w_hbm, vidx_hbm, w_hbm, v_hbm, out_hbm,
          widx_v, vidx_v, wrows, vrows, scores_v, sem):
        wid = lax.axis_index("s") * NC + lax.axis_index("c")
        lane = lax.iota(jnp.int32, LANES)

        for chunk in range(NCHUNK):
            base = wid * BPW + chunk * CB
            pltpu.sync_copy(pos_w_hbm.at[pl.ds(base, CB)], widx_v)
            for j in range(NIDX):
                pltpu.sync_copy(vidx_hbm.at[j, pl.ds(base, CB)], vidx_v.at[j])
            cps = [pltpu.async_copy(w_hbm.at[widx_v], wrows, sem)]
            for j in range(NIDX):
                cps.append(pltpu.async_copy(v_hbm.at[vidx_v.at[j]],
                                            vrows.at[j], sem))
            for cp in cps:
                cp.wait()

            for g in range(NGROUP):
                i_vec = jnp.full((LANES,), g * LANES, jnp.int32) + lane

                def body(d, accs):
                    d_vec = jnp.full((LANES,), d, jnp.int32)
                    wv = plsc.load_gather(wrows, [i_vec, d_vec])
                    return tuple(
                        accs[j] + wv * plsc.load_gather(
                            vrows,
                            [jnp.full((LANES,), j, jnp.int32), i_vec, d_vec])
                        for j in range(NIDX))

                zero = jnp.zeros((LANES,), jnp.float32)
                accs = lax.fori_loop(0, EMBED, body, (zero,) * NIDX)
                scores_v[0, pl.ds(g * LANES, LANES)] = accs[0]
                for j in range(1, NIDX):
                    scores_v[j, pl.ds(g * LANES, LANES)] = -accs[j]

            pltpu.sync_copy(scores_v, out_hbm.at[wid, chunk])

    return k(pos_w, vidx, w_table, v_table)


def _tc_loss_body(x_ref, o_ref):
    x = jnp.clip(x_ref[...], -10.0, 10.0)
    o_ref[0, 0] = -jnp.sum(jax.nn.log_sigmoid(x))


def kernel(pos_w, pos_v, neg_v, w_embeddings, v_embeddings):
    pos_w = jnp.asarray(pos_w, jnp.int32)
    vidx = jnp.concatenate(
        [jnp.asarray(pos_v, jnp.int32)[None, :],
         jnp.asarray(neg_v, jnp.int32).T], axis=0)

    scores = _sc_scores(pos_w, vidx, w_embeddings, v_embeddings)
    flat = scores.reshape(BATCH * NIDX // 128, 128)

    loss = pl.pallas_call(
        _tc_loss_body,
        out_shape=jax.ShapeDtypeStruct((1, 1), jnp.float32),
        out_specs=pl.BlockSpec(memory_space=pltpu.SMEM),
    )(flat)
    return loss[0, 0]


# speedup vs baseline: 1.5634x; 1.5634x over previous
import functools

import jax
import jax.numpy as jnp
from jax import lax
from jax.experimental import pallas as pl
from jax.experimental.pallas import tpu as pltpu
from jax.experimental.pallas import tpu_sc as plsc

VOCAB = 1000000
EMBED = 64
BATCH = 16384
NEG = 5
NIDX = NEG + 1

NC, NS, LANES = 2, 16, 16
NW = NC * NS
BPW = BATCH // NW
CB = 128
NCHUNK = BPW // CB
NGROUP = CB // LANES


def _sc_scores(pos_w, vidx, w_table, v_table):
    mesh = plsc.VectorSubcoreMesh(core_axis_name="c", subcore_axis_name="s")

    @functools.partial(
        pl.kernel,
        out_type=jax.ShapeDtypeStruct((NW, NCHUNK, NIDX, CB), jnp.float32),
        mesh=mesh,
        compiler_params=pltpu.CompilerParams(
            needs_layout_passes=False, use_tc_tiling_on_sc=False),
        scratch_types=[
            pltpu.VMEM((CB,), jnp.int32),
            pltpu.VMEM((NIDX, CB), jnp.int32),
            pltpu.VMEM((CB, EMBED), jnp.float32),
            pltpu.VMEM((NIDX, CB, EMBED), jnp.float32),
            pltpu.VMEM((NIDX, CB), jnp.float32),
            pltpu.SemaphoreType.DMA,
        ],
    )
    def k(pos_w_hbm, vidx_hbm, w_hbm, v_hbm, out_hbm,
          widx_v, vidx_v, wrows, vrows, scores_v, sem):
        wid = lax.axis_index("s") * NC + lax.axis_index("c")
        lane = lax.iota(jnp.int32, LANES)

        for chunk in range(NCHUNK):
            base = wid * BPW + chunk * CB
            pltpu.sync_copy(pos_w_hbm.at[pl.ds(base, CB)], widx_v)
            for j in range(NIDX):
                pltpu.sync_copy(vidx_hbm.at[j, pl.ds(base, CB)], vidx_v.at[j])
            cps = [pltpu.async_copy(w_hbm.at[widx_v], wrows, sem)]
            for j in range(NIDX):
                cps.append(pltpu.async_copy(v_hbm.at[vidx_v.at[j]],
                                            vrows.at[j], sem))
            for cp in cps:
                cp.wait()

            for g in range(NGROUP):
                i_vec = jnp.full((LANES,), g * LANES, jnp.int32) + lane

                def body(d, accs):
                    d_vec = jnp.full((LANES,), d, jnp.int32)
                    wv = plsc.load_gather(wrows, [i_vec, d_vec])
                    return tuple(
                        accs[j] + wv * plsc.load_gather(
                            vrows,
                            [jnp.full((LANES,), j, jnp.int32), i_vec, d_vec])
                        for j in range(NIDX))

                zero = jnp.zeros((LANES,), jnp.float32)
                accs = lax.fori_loop(0, EMBED, body, (zero,) * NIDX)
                scores_v[0, pl.ds(g * LANES, LANES)] = accs[0]
                for j in range(1, NIDX):
                    scores_v[j, pl.ds(g * LANES, LANES)] = -accs[j]

            pltpu.sync_copy(scores_v, out_hbm.at[wid, chunk])

    return k(pos_w, vidx, w_table, v_table)


def _tc_loss_body(x_ref, o_ref):
    x = jnp.clip(x_ref[...], -10.0, 10.0)
    o_ref[0, 0] = -jnp.sum(jax.nn.log_sigmoid(x))


def kernel(pos_w, pos_v, neg_v, w_embeddings, v_embeddings):
    pos_w = jnp.asarray(pos_w, jnp.int32)
    vidx = jnp.concatenate(
        [jnp.asarray(pos_v, jnp.int32)[None, :],
         jnp.asarray(neg_v, jnp.int32).T], axis=0)

    scores = _sc_scores(pos_w, vidx, w_embeddings, v_embeddings)
    flat = scores.reshape(BATCH * NIDX // 128, 128)

    loss = pl.pallas_call(
        _tc_loss_body,
        out_shape=jax.ShapeDtypeStruct((1, 1), jnp.float32),
        out_specs=pl.BlockSpec(memory_space=pltpu.SMEM),
    )(flat)
    return loss[0, 0]
